# baseline (speedup 1.0000x reference)
import jax
import jax.numpy as jnp
from jax import lax
from jax.experimental import pallas as pl
from jax.experimental.pallas import tpu as pltpu
from jax.experimental.pallas import tpu_sc as plsc

BATCH = 16
SEQ_LEN = 4096
DIM = 256
P = 1024

NC = 2
NS = 16
LANES = 16
NW = NC * NS
PATCH_PER_W = (BATCH * P) // NW
WIN = 64
NSUB = PATCH_PER_W // WIN
ACC_ROWS = WIN + 16
CHUNK = 128
NCHUNKS = SEQ_LEN // CHUNK
IDS_PAD = SEQ_LEN + LANES


def _sc_body(h_hbm, pid_hbm, out_hbm, ids_l, hbuf0, hbuf1, acc, cacc,
             obuf, sem0, sem1, semo):
    c = lax.axis_index("c")
    s = lax.axis_index("s")
    wid = s * NC + c
    row = wid // 2
    half = wid % 2

    pltpu.sync_copy(pid_hbm.at[row], ids_l)

    onev = jnp.ones((LANES,), jnp.float32)
    zerov = jnp.zeros((LANES,), jnp.float32)

    def id_at(t):
        return ids_l[pl.ds(t, LANES)][0]

    def lower_bound(target):
        lo = jnp.int32(0)
        sh = SEQ_LEN // 2
        while sh >= 1:
            below = id_at(lo + (sh - 1)) < target
            lo = jnp.where(below, lo + sh, lo)
            sh //= 2
        return lo

    def window(sub, t_lo):
        p0 = half * PATCH_PER_W + sub * WIN
        t_hi = lower_bound(p0 + WIN)
        j0 = t_lo // CHUNK
        j1 = jnp.maximum((t_hi + CHUNK - 1) // CHUNK, j0 + 1)

        def start(j, buf, sem):
            pltpu.async_copy(h_hbm.at[row, pl.ds(j * CHUNK, CHUNK)], buf, sem)

        def wait(j, buf, sem):
            pltpu.make_async_copy(
                h_hbm.at[row, pl.ds(j * CHUNK, CHUNK)], buf, sem).wait()

        def flush(prev, cnt, run):
            for k in range(DIM // LANES):
                sl = pl.ds(k * LANES, LANES)
                acc[prev, sl] = acc[prev, sl] + run[k]
            cacc[prev] = cacc[prev] + cnt

        def compute(j, buf):
            g0 = jnp.clip((t_lo - j * CHUNK) // LANES, 0, CHUNK // LANES)
            g1 = jnp.clip((t_hi - j * CHUNK + LANES - 1) // LANES,
                          0, CHUNK // LANES)

            def group(g, carry):
                prev, cnt = carry[0], carry[1]
                run = list(carry[2:])
                idv = ids_l[pl.ds(j * CHUNK + g * LANES, LANES)]
                lpv = jnp.clip(idv - p0, -1, WIN) + 8
                for u in range(LANES):
                    lp = lpv[u]
                    t = g * LANES + u
                    change = lp != prev

                    @pl.when(change)
                    def _(prev=prev, cnt=cnt, run=tuple(run)):
                        flush(prev, cnt, run)

                    keep = jnp.where(change, 0.0, 1.0)
                    for k in range(DIM // LANES):
                        run[k] = run[k] * keep + buf[t, pl.ds(k * LANES,
                                                              LANES)]
                    cnt = cnt * keep + 1.0
                    prev = lp
                return (prev, cnt, *run)

            init = (jnp.int32(0), zerov) + (zerov,) * (DIM // LANES)
            fin = lax.fori_loop(g0, g1, group, init)
            flush(fin[0], fin[1], fin[2:])

        def pair(jj, _):
            a = 2 * jj
            b = a + 1
            in_a = (a >= j0) & (a < j1)
            in_b = (b >= j0) & (b < j1)

            @pl.when(in_b & (b > j0))
            def _():
                start(b, hbuf1, sem1)

            @pl.when(in_a)
            def _():
                wait(a, hbuf0, sem0)
                compute(a, hbuf0)

            @pl.when((a + 2 > j0) & (a + 2 < j1))
            def _():
                start(a + 2, hbuf0, sem0)

            @pl.when(in_b)
            def _():
                wait(b, hbuf1, sem1)
                compute(b, hbuf1)

            return 0

        lax.fori_loop(j0 // 2, (j1 + 1) // 2, pair, 0)

        nj0 = t_hi // CHUNK

        @pl.when((sub + 1 < NSUB) & (nj0 % 2 == 0))
        def _():
            start(nj0, hbuf0, sem0)

        @pl.when((sub + 1 < NSUB) & (nj0 % 2 == 1))
        def _():
            start(nj0, hbuf1, sem1)

        @pl.when(sub > 0)
        def _():
            pltpu.make_async_copy(
                obuf, out_hbm.at[row, pl.ds(p0 - WIN, WIN)], semo).wait()

        def normalize(i, _):
            inv = 1.0 / jnp.maximum(cacc[i + 8], 1.0)
            for k in range(DIM // LANES):
                sl = pl.ds(k * LANES, LANES)
                obuf[i, sl] = acc[i + 8, sl] * inv
                acc[i + 8, sl] = zerov
            cacc[i + 8] = zerov
            return 0

        lax.fori_loop(0, WIN, normalize, 0)
        pltpu.async_copy(obuf, out_hbm.at[row, pl.ds(p0, WIN)], semo)
        return t_hi

    def zero(i, _):
        for k in range(DIM // LANES):
            acc[i, pl.ds(k * LANES, LANES)] = zerov
        cacc[i] = zerov
        return 0

    lax.fori_loop(0, ACC_ROWS, zero, 0)

    t_first = lower_bound(half * PATCH_PER_W)
    jf = t_first // CHUNK

    @pl.when(jf % 2 == 0)
    def _():
        pltpu.async_copy(h_hbm.at[row, pl.ds(jf * CHUNK, CHUNK)], hbuf0, sem0)

    @pl.when(jf % 2 == 1)
    def _():
        pltpu.async_copy(h_hbm.at[row, pl.ds(jf * CHUNK, CHUNK)], hbuf1, sem1)

    lax.fori_loop(0, NSUB, window, t_first)
    last_p0 = half * PATCH_PER_W + (NSUB - 1) * WIN
    pltpu.make_async_copy(
        obuf, out_hbm.at[row, pl.ds(last_p0, WIN)], semo).wait()


@jax.jit
def kernel(h, patch_ids):
    pid = patch_ids.astype(jnp.int32)
    pid = jnp.pad(pid, ((0, 0), (0, IDS_PAD - SEQ_LEN)), mode="edge")

    run = pl.kernel(
        _sc_body,
        out_type=jax.ShapeDtypeStruct((BATCH, P, DIM), jnp.float32),
        mesh=plsc.VectorSubcoreMesh(core_axis_name="c", subcore_axis_name="s"),
        scratch_types=[
            pltpu.VMEM((IDS_PAD,), jnp.int32),
            pltpu.VMEM((CHUNK, DIM), jnp.float32),
            pltpu.VMEM((CHUNK, DIM), jnp.float32),
            pltpu.VMEM((ACC_ROWS, DIM), jnp.float32),
            pltpu.VMEM((ACC_ROWS, LANES), jnp.float32),
            pltpu.VMEM((WIN, DIM), jnp.float32),
            pltpu.SemaphoreType.DMA,
            pltpu.SemaphoreType.DMA,
            pltpu.SemaphoreType.DMA,
        ],
    )
    return run(h, pid)


# speedup vs baseline: 1.2046x; 1.2046x over previous
import jax
import jax.numpy as jnp
from jax import lax
from jax.experimental import pallas as pl
from jax.experimental.pallas import tpu as pltpu
from jax.experimental.pallas import tpu_sc as plsc

BATCH = 16
SEQ_LEN = 4096
DIM = 256
P = 1024

NC = 2
NS = 16
LANES = 16
NW = NC * NS
PATCH_PER_W = (BATCH * P) // NW
WIN = 128
NSUB = PATCH_PER_W // WIN
ACC_ROWS = WIN + 16
CHUNK = 64
NCHUNKS = SEQ_LEN // CHUNK
IDS_PAD = SEQ_LEN + LANES


def _sc_body(h_hbm, pid_hbm, out_hbm, ids_l, hbuf0, hbuf1, acc, cacc,
             sem0, sem1):
    c = lax.axis_index("c")
    s = lax.axis_index("s")
    wid = s * NC + c
    row = wid // 2
    half = wid % 2

    pltpu.sync_copy(pid_hbm.at[row], ids_l)

    onev = jnp.ones((LANES,), jnp.float32)
    zerov = jnp.zeros((LANES,), jnp.float32)

    def id_at(t):
        return ids_l[pl.ds(t, LANES)][0]

    def lower_bound(target):
        lo = jnp.int32(0)
        sh = SEQ_LEN // 2
        while sh >= 1:
            below = id_at(lo + (sh - 1)) < target
            lo = jnp.where(below, lo + sh, lo)
            sh //= 2
        return lo

    def window(sub, t_lo):
        p0 = half * PATCH_PER_W + sub * WIN
        t_hi = lower_bound(p0 + WIN)
        j0 = t_lo // CHUNK
        j1 = jnp.maximum((t_hi + CHUNK - 1) // CHUNK, j0 + 1)

        def zero(i, _):
            for k in range(DIM // LANES):
                acc[i, pl.ds(k * LANES, LANES)] = zerov
            cacc[i] = zerov
            return 0

        lax.fori_loop(0, ACC_ROWS, zero, 0)

        def start(j, buf, sem):
            pltpu.async_copy(h_hbm.at[row, pl.ds(j * CHUNK, CHUNK)], buf, sem)

        def wait(j, buf, sem):
            pltpu.make_async_copy(
                h_hbm.at[row, pl.ds(j * CHUNK, CHUNK)], buf, sem).wait()

        def flush(prev, cnt, run):
            for k in range(DIM // LANES):
                sl = pl.ds(k * LANES, LANES)
                acc[prev, sl] = acc[prev, sl] + run[k]
            cacc[prev] = cacc[prev] + cnt

        def compute(j, buf):
            g0 = jnp.clip((t_lo - j * CHUNK) // LANES, 0, CHUNK // LANES)
            g1 = jnp.clip((t_hi - j * CHUNK + LANES - 1) // LANES,
                          0, CHUNK // LANES)

            def group(g, carry):
                prev, cnt = carry[0], carry[1]
                run = list(carry[2:])
                idv = ids_l[pl.ds(j * CHUNK + g * LANES, LANES)]
                lpv = jnp.clip(idv - p0, -1, WIN) + 8
                for u in range(LANES):
                    lp = lpv[u]
                    t = g * LANES + u
                    change = lp != prev

                    @pl.when(change)
                    def _(prev=prev, cnt=cnt, run=tuple(run)):
                        flush(prev, cnt, run)

                    keep = jnp.where(change, 0.0, 1.0)
                    for k in range(DIM // LANES):
                        run[k] = run[k] * keep + buf[t, pl.ds(k * LANES,
                                                              LANES)]
                    cnt = cnt * keep + 1.0
                    prev = lp
                return (prev, cnt, *run)

            init = (jnp.int32(0), zerov) + (zerov,) * (DIM // LANES)
            fin = lax.fori_loop(g0, g1, group, init)
            flush(fin[0], fin[1], fin[2:])

        def pair(jj, _):
            a = 2 * jj
            b = a + 1
            in_a = (a >= j0) & (a < j1)
            in_b = (b >= j0) & (b < j1)

            @pl.when(in_b & (b > j0))
            def _():
                start(b, hbuf1, sem1)

            @pl.when(in_a)
            def _():
                wait(a, hbuf0, sem0)
                compute(a, hbuf0)

            @pl.when((a + 2 > j0) & (a + 2 < j1))
            def _():
                start(a + 2, hbuf0, sem0)

            @pl.when(in_b)
            def _():
                wait(b, hbuf1, sem1)
                compute(b, hbuf1)

            return 0

        lax.fori_loop(j0 // 2, (j1 + 1) // 2, pair, 0)

        nj0 = t_hi // CHUNK

        @pl.when((sub + 1 < NSUB) & (nj0 % 2 == 0))
        def _():
            start(nj0, hbuf0, sem0)

        @pl.when((sub + 1 < NSUB) & (nj0 % 2 == 1))
        def _():
            start(nj0, hbuf1, sem1)

        def normalize(i, _):
            inv = 1.0 / jnp.maximum(cacc[i], 1.0)
            for k in range(DIM // LANES):
                sl = pl.ds(k * LANES, LANES)
                acc[i, sl] = acc[i, sl] * inv
            return 0

        lax.fori_loop(8, WIN + 8, normalize, 0)
        pltpu.sync_copy(acc.at[pl.ds(8, WIN)], out_hbm.at[row, pl.ds(p0, WIN)])
        return t_hi

    t_first = lower_bound(half * PATCH_PER_W)
    jf = t_first // CHUNK

    @pl.when(jf % 2 == 0)
    def _():
        pltpu.async_copy(h_hbm.at[row, pl.ds(jf * CHUNK, CHUNK)], hbuf0, sem0)

    @pl.when(jf % 2 == 1)
    def _():
        pltpu.async_copy(h_hbm.at[row, pl.ds(jf * CHUNK, CHUNK)], hbuf1, sem1)

    lax.fori_loop(0, NSUB, window, t_first)


@jax.jit
def kernel(h, patch_ids):
    pid = patch_ids.astype(jnp.int32)
    pid = jnp.pad(pid, ((0, 0), (0, IDS_PAD - SEQ_LEN)), mode="edge")

    run = pl.kernel(
        _sc_body,
        out_type=jax.ShapeDtypeStruct((BATCH, P, DIM), jnp.float32),
        mesh=plsc.VectorSubcoreMesh(core_axis_name="c", subcore_axis_name="s"),
        scratch_types=[
            pltpu.VMEM((IDS_PAD,), jnp.int32),
            pltpu.VMEM((CHUNK, DIM), jnp.float32),
            pltpu.VMEM((CHUNK, DIM), jnp.float32),
            pltpu.VMEM((ACC_ROWS, DIM), jnp.float32),
            pltpu.VMEM((ACC_ROWS, LANES), jnp.float32),
            pltpu.SemaphoreType.DMA,
            pltpu.SemaphoreType.DMA,
        ],
    )
    return run(h, pid)


# speedup vs baseline: 1.3481x; 1.1191x over previous
import jax
import jax.numpy as jnp
from jax import lax
from jax.experimental import pallas as pl
from jax.experimental.pallas import tpu as pltpu
from jax.experimental.pallas import tpu_sc as plsc

BATCH = 16
SEQ_LEN = 4096
DIM = 256
P = 1024

NC = 2
NS = 16
LANES = 16
NW = NC * NS
PATCH_PER_W = (BATCH * P) // NW
WIN = 128
NSUB = PATCH_PER_W // WIN
ACC_ROWS = WIN + 16
CHUNK = 128
NCHUNKS = SEQ_LEN // CHUNK
IDS_PAD = SEQ_LEN + LANES


def _sc_body(h_hbm, pid_hbm, out_hbm, ids_l, hbuf0, hbuf1, acc, cacc,
             sem0, sem1):
    c = lax.axis_index("c")
    s = lax.axis_index("s")
    wid = s * NC + c
    row = wid // 2
    half = wid % 2

    pltpu.sync_copy(pid_hbm.at[row], ids_l)

    onev = jnp.ones((LANES,), jnp.float32)
    zerov = jnp.zeros((LANES,), jnp.float32)

    def id_at(t):
        return ids_l[pl.ds(t, LANES)][0]

    def lower_bound(target):
        lo = jnp.int32(0)
        sh = SEQ_LEN // 2
        while sh >= 1:
            below = id_at(lo + (sh - 1)) < target
            lo = jnp.where(below, lo + sh, lo)
            sh //= 2
        return lo

    def window(sub, t_lo):
        p0 = half * PATCH_PER_W + sub * WIN
        t_hi = lower_bound(p0 + WIN)
        j0 = t_lo // CHUNK
        j1 = jnp.maximum((t_hi + CHUNK - 1) // CHUNK, j0 + 2)

        def zero(i, _):
            for k in range(DIM // LANES):
                acc[i, pl.ds(k * LANES, LANES)] = zerov
            cacc[i] = zerov
            return 0

        lax.fori_loop(0, ACC_ROWS, zero, 0)

        def start(j, buf, sem):
            jc = jnp.minimum(j, NCHUNKS - 1)
            pltpu.async_copy(h_hbm.at[row, pl.ds(jc * CHUNK, CHUNK)], buf, sem)

        def wait(j, buf, sem):
            jc = jnp.minimum(j, NCHUNKS - 1)
            pltpu.make_async_copy(
                h_hbm.at[row, pl.ds(jc * CHUNK, CHUNK)], buf, sem).wait()

        def flush(prev, cnt, run):
            for k in range(DIM // LANES):
                sl = pl.ds(k * LANES, LANES)
                acc[prev, sl] = acc[prev, sl] + run[k]
            cacc[prev] = cacc[prev] + cnt

        def compute(j, buf):
            g0 = jnp.clip((t_lo - j * CHUNK) // LANES, 0, CHUNK // LANES)
            g1 = jnp.clip((t_hi - j * CHUNK + LANES - 1) // LANES,
                          0, CHUNK // LANES)

            def group(g, carry):
                prev, cnt = carry[0], carry[1]
                run = list(carry[2:])
                idv = ids_l[pl.ds(j * CHUNK + g * LANES, LANES)]
                lpv = jnp.clip(idv - p0, -1, WIN) + 8
                for u in range(LANES):
                    lp = lpv[u]
                    t = g * LANES + u
                    change = lp != prev

                    @pl.when(change)
                    def _(prev=prev, cnt=cnt, run=tuple(run)):
                        flush(prev, cnt, run)

                    keep = jnp.where(change, 0.0, 1.0)
                    for k in range(DIM // LANES):
                        run[k] = run[k] * keep + buf[t, pl.ds(k * LANES,
                                                              LANES)]
                    cnt = cnt * keep + 1.0
                    prev = lp
                return (prev, cnt, *run)

            init = (jnp.int32(0), zerov) + (zerov,) * (DIM // LANES)
            fin = lax.fori_loop(g0, g1, group, init)
            flush(fin[0], fin[1], fin[2:])

        def pair(jj, _):
            a = 2 * jj
            b = a + 1
            in_a = (a >= j0) & (a < j1)
            in_b = (b >= j0) & (b < j1)

            @pl.when(in_b & (b > j0 + 1))
            def _():
                start(b, hbuf1, sem1)

            @pl.when(in_a)
            def _():
                wait(a, hbuf0, sem0)
                compute(a, hbuf0)

            @pl.when((a + 2 > j0 + 1) & (a + 2 < j1))
            def _():
                start(a + 2, hbuf0, sem0)

            @pl.when(in_b)
            def _():
                wait(b, hbuf1, sem1)
                compute(b, hbuf1)

            return 0

        lax.fori_loop(j0 // 2, (j1 + 1) // 2, pair, 0)

        nj0 = t_hi // CHUNK

        @pl.when((sub + 1 < NSUB) & (nj0 % 2 == 0))
        def _():
            start(nj0, hbuf0, sem0)
            start(nj0 + 1, hbuf1, sem1)

        @pl.when((sub + 1 < NSUB) & (nj0 % 2 == 1))
        def _():
            start(nj0, hbuf1, sem1)
            start(nj0 + 1, hbuf0, sem0)

        def normalize(i, _):
            inv = 1.0 / jnp.maximum(cacc[i], 1.0)
            for k in range(DIM // LANES):
                sl = pl.ds(k * LANES, LANES)
                acc[i, sl] = acc[i, sl] * inv
            return 0

        lax.fori_loop(8, WIN + 8, normalize, 0)
        pltpu.sync_copy(acc.at[pl.ds(8, WIN)], out_hbm.at[row, pl.ds(p0, WIN)])
        return t_hi

    t_first = lower_bound(half * PATCH_PER_W)
    jf = t_first // CHUNK

    jg = jnp.minimum(jf + 1, NCHUNKS - 1)

    @pl.when(jf % 2 == 0)
    def _():
        pltpu.async_copy(h_hbm.at[row, pl.ds(jf * CHUNK, CHUNK)], hbuf0, sem0)
        pltpu.async_copy(h_hbm.at[row, pl.ds(jg * CHUNK, CHUNK)], hbuf1, sem1)

    @pl.when(jf % 2 == 1)
    def _():
        pltpu.async_copy(h_hbm.at[row, pl.ds(jf * CHUNK, CHUNK)], hbuf1, sem1)
        pltpu.async_copy(h_hbm.at[row, pl.ds(jg * CHUNK, CHUNK)], hbuf0, sem0)

    lax.fori_loop(0, NSUB, window, t_first)


@jax.jit
def kernel(h, patch_ids):
    pid = patch_ids.astype(jnp.int32)
    pid = jnp.pad(pid, ((0, 0), (0, IDS_PAD - SEQ_LEN)), mode="edge")

    run = pl.kernel(
        _sc_body,
        out_type=jax.ShapeDtypeStruct((BATCH, P, DIM), jnp.float32),
        mesh=plsc.VectorSubcoreMesh(core_axis_name="c", subcore_axis_name="s"),
        scratch_types=[
            pltpu.VMEM((IDS_PAD,), jnp.int32),
            pltpu.VMEM((CHUNK, DIM), jnp.float32),
            pltpu.VMEM((CHUNK, DIM), jnp.float32),
            pltpu.VMEM((ACC_ROWS, DIM), jnp.float32),
            pltpu.VMEM((ACC_ROWS, LANES), jnp.float32),
            pltpu.SemaphoreType.DMA,
            pltpu.SemaphoreType.DMA,
        ],
    )
    return run(h, pid)


# speedup vs baseline: 1.4195x; 1.0529x over previous
import jax
import jax.numpy as jnp
from jax import lax
from jax.experimental import pallas as pl
from jax.experimental.pallas import tpu as pltpu
from jax.experimental.pallas import tpu_sc as plsc

BATCH = 16
SEQ_LEN = 4096
DIM = 256
P = 1024

NC = 2
NS = 16
LANES = 16
NW = NC * NS
PATCH_PER_W = (BATCH * P) // NW
WIN = 128
NSUB = PATCH_PER_W // WIN
ACC_ROWS = WIN + 16
CHUNK = 128
NCHUNKS = SEQ_LEN // CHUNK
IDS_PAD = SEQ_LEN + LANES


def _sc_body(h_hbm, pid_hbm, out_hbm, ids_l, hbuf0, hbuf1, acc, cacc,
             sem0, sem1):
    c = lax.axis_index("c")
    s = lax.axis_index("s")
    wid = s * NC + c
    row = wid // 2
    half = wid % 2

    pltpu.sync_copy(pid_hbm.at[row], ids_l)

    onev = jnp.ones((LANES,), jnp.float32)
    zerov = jnp.zeros((LANES,), jnp.float32)

    def id_at(t):
        return ids_l[pl.ds(t, LANES)][0]

    def lower_bound(target):
        lo = jnp.int32(0)
        sh = SEQ_LEN // 2
        while sh >= 1:
            below = id_at(lo + (sh - 1)) < target
            lo = jnp.where(below, lo + sh, lo)
            sh //= 2
        return lo

    def window(sub, t_lo):
        p0 = half * PATCH_PER_W + sub * WIN
        t_hi = lower_bound(p0 + WIN)
        j0 = t_lo // CHUNK
        j1 = jnp.maximum((t_hi + CHUNK - 1) // CHUNK, j0 + 2)

        def zero(i, _):
            for k in range(DIM // LANES):
                acc[pl.ds(i * DIM + k * LANES, LANES)] = zerov
            cacc[pl.ds(i * LANES, LANES)] = zerov
            return 0

        lax.fori_loop(0, ACC_ROWS, zero, 0)

        def start(j, buf, sem):
            jc = jnp.minimum(j, NCHUNKS - 1)
            pltpu.async_copy(h_hbm.at[row, pl.ds(jc * CHUNK, CHUNK)], buf, sem)

        def wait(j, buf, sem):
            jc = jnp.minimum(j, NCHUNKS - 1)
            pltpu.make_async_copy(
                h_hbm.at[row, pl.ds(jc * CHUNK, CHUNK)], buf, sem).wait()

        def flush(prev, cnt, run):
            base = prev * DIM
            for k in range(DIM // LANES):
                plsc.addupdate(acc.at[pl.ds(base + k * LANES, LANES)], run[k])
            plsc.addupdate(cacc.at[pl.ds(prev * LANES, LANES)], cnt)

        def compute(j, buf):
            g0 = jnp.clip((t_lo - j * CHUNK) // LANES, 0, CHUNK // LANES)
            g1 = jnp.clip((t_hi - j * CHUNK + LANES - 1) // LANES,
                          0, CHUNK // LANES)

            def group(g, carry):
                prev, cnt = carry[0], carry[1]
                run = list(carry[2:])
                idv = ids_l[pl.ds(j * CHUNK + g * LANES, LANES)]
                lpv = jnp.clip(idv - p0, -1, WIN) + 8
                for u in range(LANES):
                    lp = lpv[u]
                    t = g * LANES + u
                    change = lp != prev

                    @pl.when(change)
                    def _(prev=prev, cnt=cnt, run=tuple(run)):
                        flush(prev, cnt, run)

                    keep = jnp.where(change, 0.0, 1.0)
                    for k in range(DIM // LANES):
                        run[k] = run[k] * keep + buf[t, pl.ds(k * LANES,
                                                              LANES)]
                    cnt = cnt * keep + 1.0
                    prev = lp
                return (prev, cnt, *run)

            init = (jnp.int32(0), zerov) + (zerov,) * (DIM // LANES)
            fin = lax.fori_loop(g0, g1, group, init)
            flush(fin[0], fin[1], fin[2:])

        def pair(jj, _):
            a = 2 * jj
            b = a + 1
            in_a = (a >= j0) & (a < j1)
            in_b = (b >= j0) & (b < j1)

            @pl.when(in_b & (b > j0 + 1))
            def _():
                start(b, hbuf1, sem1)

            @pl.when(in_a)
            def _():
                wait(a, hbuf0, sem0)
                compute(a, hbuf0)

            @pl.when((a + 2 > j0 + 1) & (a + 2 < j1))
            def _():
                start(a + 2, hbuf0, sem0)

            @pl.when(in_b)
            def _():
                wait(b, hbuf1, sem1)
                compute(b, hbuf1)

            return 0

        lax.fori_loop(j0 // 2, (j1 + 1) // 2, pair, 0)

        nj0 = t_hi // CHUNK

        @pl.when((sub + 1 < NSUB) & (nj0 % 2 == 0))
        def _():
            start(nj0, hbuf0, sem0)
            start(nj0 + 1, hbuf1, sem1)

        @pl.when((sub + 1 < NSUB) & (nj0 % 2 == 1))
        def _():
            start(nj0, hbuf1, sem1)
            start(nj0 + 1, hbuf0, sem0)

        def normalize(i, _):
            cnt = cacc[pl.ds(i * LANES, LANES)]
            inv = 1.0 / jnp.maximum(cnt, 1.0)
            for k in range(DIM // LANES):
                sl = pl.ds(i * DIM + k * LANES, LANES)
                acc[sl] = acc[sl] * inv
            return 0

        lax.fori_loop(8, WIN + 8, normalize, 0)
        pltpu.sync_copy(acc.at[pl.ds(8 * DIM, WIN * DIM)],
                        out_hbm.at[row, pl.ds(p0 * DIM, WIN * DIM)])
        return t_hi

    t_first = lower_bound(half * PATCH_PER_W)
    jf = t_first // CHUNK

    jg = jnp.minimum(jf + 1, NCHUNKS - 1)

    @pl.when(jf % 2 == 0)
    def _():
        pltpu.async_copy(h_hbm.at[row, pl.ds(jf * CHUNK, CHUNK)], hbuf0, sem0)
        pltpu.async_copy(h_hbm.at[row, pl.ds(jg * CHUNK, CHUNK)], hbuf1, sem1)

    @pl.when(jf % 2 == 1)
    def _():
        pltpu.async_copy(h_hbm.at[row, pl.ds(jf * CHUNK, CHUNK)], hbuf1, sem1)
        pltpu.async_copy(h_hbm.at[row, pl.ds(jg * CHUNK, CHUNK)], hbuf0, sem0)

    lax.fori_loop(0, NSUB, window, t_first)


@jax.jit
def kernel(h, patch_ids):
    pid = patch_ids.astype(jnp.int32)
    pid = jnp.pad(pid, ((0, 0), (0, IDS_PAD - SEQ_LEN)), mode="edge")

    run = pl.kernel(
        _sc_body,
        out_type=jax.ShapeDtypeStruct((BATCH, P * DIM), jnp.float32),
        mesh=plsc.VectorSubcoreMesh(core_axis_name="c", subcore_axis_name="s"),
        scratch_types=[
            pltpu.VMEM((IDS_PAD,), jnp.int32),
            pltpu.VMEM((CHUNK, DIM), jnp.float32),
            pltpu.VMEM((CHUNK, DIM), jnp.float32),
            pltpu.VMEM((ACC_ROWS * DIM,), jnp.float32),
            pltpu.VMEM((ACC_ROWS * LANES,), jnp.float32),
            pltpu.SemaphoreType.DMA,
            pltpu.SemaphoreType.DMA,
        ],
    )
    return run(h, pid).reshape(BATCH, P, DIM)
